# baseline (speedup 1.0000x reference)
import functools

import jax
import jax.numpy as jnp
from jax import lax
from jax.experimental import pallas as pl
from jax.experimental.pallas import tpu as pltpu
from jax.experimental.pallas import tpu_sc as plsc

N = 10000
E = 320000
D_IN = 128
D_HID = 128
N_CLS = 64

L = 16
NSC = 2
NTILE = 16
NW = NSC * NTILE
EPT = E // NW
C = 80
NCH = EPT // C
NPAD = 10240
RPT = NPAD // NTILE

_mesh = plsc.VectorSubcoreMesh(core_axis_name="c", subcore_axis_name="s")
_sc_params = pltpu.CompilerParams(needs_layout_passes=False,
                                  use_tc_tiling_on_sc=False)


def _sc_deg_body(dst_hbm, ew_hbm, out_hbm, dst_v, ew_v, acc, zbuf):
    c = lax.axis_index("c")
    s = lax.axis_index("s")
    wid = s * NSC + c
    pltpu.sync_copy(dst_hbm.at[wid], dst_v)
    pltpu.sync_copy(ew_hbm.at[wid], ew_v)

    zeros = jnp.zeros((L,), jnp.float32)

    def zb(i, _):
        zbuf[pl.ds(i * L, L)] = zeros
        return 0

    lax.fori_loop(0, RPT // L, zb, 0)
    pltpu.sync_copy(zbuf, acc.at[pl.ds(s * RPT, RPT)])
    plsc.subcore_barrier()

    def chunk(j, _):
        pltpu.sync_copy(ew_v.at[j], acc.at[dst_v.at[j]], add=True)
        return 0

    lax.fori_loop(0, NCH, chunk, 0)
    plsc.subcore_barrier()
    pltpu.sync_copy(acc.at[pl.ds(s * RPT, RPT)],
                    out_hbm.at[c].at[pl.ds(s * RPT, RPT)])


def _sc_deg(dst_r, ew_r):
    return pl.kernel(
        _sc_deg_body,
        out_type=jax.ShapeDtypeStruct((NSC, NPAD), jnp.float32),
        mesh=_mesh,
        scratch_types=[
            pltpu.VMEM((NCH, C), jnp.int32),
            pltpu.VMEM((NCH, C), jnp.float32),
            pltpu.VMEM_SHARED((NPAD,), jnp.float32),
            pltpu.VMEM((RPT,), jnp.float32),
        ],
        compiler_params=_sc_params,
    )(dst_r, ew_r)


def _sc_prop_body(h, g_hbm, src_hbm, dst_hbm, ew_hbm, out_hbm,
                  src_v, dst_v, ew_v, rows0, rows1, sc0, sc1, acc,
                  sem0, sem1, ssem0, ssem1):
    D = 64
    c = lax.axis_index("c")
    s = lax.axis_index("s")
    wid = s * NSC + c
    pltpu.sync_copy(src_hbm.at[wid], src_v)
    pltpu.sync_copy(dst_hbm.at[wid], dst_v)
    pltpu.sync_copy(ew_hbm.at[wid], ew_v)

    zeros = jnp.zeros((L,), jnp.float32)

    def zrow(r, _):
        for k in range(D // L):
            sc0[r, pl.ds(k * L, L)] = zeros
        return 0

    lax.fori_loop(0, C, zrow, 0)
    for i in range(RPT // C):
        pltpu.sync_copy(sc0, acc.at[pl.ds(s * RPT + i * C, C)])
    plsc.subcore_barrier()

    def scale(j, rows, sc):
        def group(t, _):
            base = jnp.full((L,), j * C + t * L, jnp.int32)
            for i in range(L):
                ewb = plsc.load_gather(ew_v, [base + i])
                e = t * L + i
                for k in range(D // L):
                    sc[e, pl.ds(k * L, L)] = \
                        rows[e, pl.ds(h * D + k * L, L)] * ewb
            return 0

        lax.fori_loop(0, C // L, group, 0)

    def gather(j, rows, sem):
        return pltpu.async_copy(g_hbm.at[src_v.at[j]], rows, sem)

    gather(0, rows0, sem0)

    def step(j, rows, sem, nrows, nsem, sc, ssem, last):
        pltpu.make_async_copy(g_hbm.at[src_v.at[j]], rows, sem).wait()
        if not last:
            gather(j + 1, nrows, nsem)

        @pl.when(j >= 2)
        def _():
            pltpu.make_async_copy(sc, acc.at[dst_v.at[j]], ssem).wait()

        scale(j, rows, sc)
        pltpu.async_copy(sc, acc.at[dst_v.at[j]], ssem, add=True)

    def chunk2(jj, _):
        for b in range(2):
            j = jj * 2 + b
            rows, sem, sc, ssem = ((rows0, sem0, sc0, ssem0) if b == 0
                                   else (rows1, sem1, sc1, ssem1))
            nrows, nsem = (rows1, sem1) if b == 0 else (rows0, sem0)
            step(j, rows, sem, nrows, nsem, sc, ssem, False)
        return 0

    lax.fori_loop(0, NCH // 2, chunk2, 0)
    jt = NCH - 1
    step(jt, rows0, sem0, rows1, sem1, sc0, ssem0, True)
    pltpu.make_async_copy(sc1, acc.at[dst_v.at[jt]], ssem1).wait()
    pltpu.make_async_copy(sc0, acc.at[dst_v.at[jt]], ssem0).wait()

    plsc.subcore_barrier()
    for i in range(RPT // C):
        sl = pl.ds(s * RPT + i * C, C)
        pltpu.sync_copy(acc.at[sl], out_hbm.at[c].at[sl])


def _sc_prop(g, src_r, dst_r, ew_flat, h):
    D = 64
    return pl.kernel(
        functools.partial(_sc_prop_body, h),
        out_type=jax.ShapeDtypeStruct((NSC, NPAD, D), jnp.float32),
        mesh=_mesh,
        scratch_types=[
            pltpu.VMEM((NCH, C), jnp.int32),
            pltpu.VMEM((NCH, C), jnp.int32),
            pltpu.VMEM((EPT,), jnp.float32),
            pltpu.VMEM((C, 128), jnp.float32),
            pltpu.VMEM((C, 128), jnp.float32),
            pltpu.VMEM((C, D), jnp.float32),
            pltpu.VMEM((C, D), jnp.float32),
            pltpu.VMEM_SHARED((NPAD, D), jnp.float32),
            pltpu.SemaphoreType.DMA,
            pltpu.SemaphoreType.DMA,
            pltpu.SemaphoreType.DMA,
            pltpu.SemaphoreType.DMA,
        ],
        compiler_params=_sc_params,
    )(g, src_r, dst_r, ew_flat)


BLK = 1000


def _dis_from(dp_ref):
    return lax.rsqrt(1.0 + dp_ref[:, 0:1] + dp_ref[:, 1:2])


def _tc_mm1_body(x_ref, w_ref, dp_ref, g_ref):
    dis = _dis_from(dp_ref)
    h = jnp.dot(x_ref[...], w_ref[...], preferred_element_type=jnp.float32)
    g_ref[...] = h * dis


def _tc_mid_body(pl_ref, pr_ref, g1_ref, dp_ref, b1_ref, w2_ref, g2_ref):
    dis = _dis_from(dp_ref)
    m = jnp.concatenate(
        [pl_ref[0, :, :] + pl_ref[1, :, :], pr_ref[0, :, :] + pr_ref[1, :, :]],
        axis=1)
    z = dis * (m + g1_ref[...]) + b1_ref[...]
    z = jnp.maximum(z, 0.0)
    h2 = jnp.dot(z, w2_ref[...], preferred_element_type=jnp.float32)
    g2_ref[...] = jnp.concatenate(
        [h2 * dis, jnp.zeros_like(h2)], axis=1)


def _tc_final_body(q_ref, g2_ref, dp_ref, b2_ref, o_ref):
    dis = _dis_from(dp_ref)
    o_ref[...] = dis * (q_ref[0, :, :] + q_ref[1, :, :]
                        + g2_ref[:, :N_CLS]) + b2_ref[...]


def _tc_mm1(x, W1, dp_t):
    grid = (N // BLK,)
    return pl.pallas_call(
        _tc_mm1_body,
        grid=grid,
        in_specs=[
            pl.BlockSpec((BLK, D_IN), lambda i: (i, 0)),
            pl.BlockSpec((D_IN, D_HID), lambda i: (0, 0)),
            pl.BlockSpec((BLK, 2), lambda i: (i, 0)),
        ],
        out_specs=pl.BlockSpec((BLK, D_HID), lambda i: (i, 0)),
        out_shape=jax.ShapeDtypeStruct((N, D_HID), jnp.float32),
    )(x, W1, dp_t)


def _tc_mid(p_l, p_r, g1, dp_t, b1, W2):
    grid = (N // BLK,)
    half = D_HID // 2
    return pl.pallas_call(
        _tc_mid_body,
        grid=grid,
        in_specs=[
            pl.BlockSpec((NSC, BLK, half), lambda i: (0, i, 0)),
            pl.BlockSpec((NSC, BLK, half), lambda i: (0, i, 0)),
            pl.BlockSpec((BLK, D_HID), lambda i: (i, 0)),
            pl.BlockSpec((BLK, 2), lambda i: (i, 0)),
            pl.BlockSpec((1, D_HID), lambda i: (0, 0)),
            pl.BlockSpec((D_HID, N_CLS), lambda i: (0, 0)),
        ],
        out_specs=pl.BlockSpec((BLK, 2 * N_CLS), lambda i: (i, 0)),
        out_shape=jax.ShapeDtypeStruct((N, 2 * N_CLS), jnp.float32),
    )(p_l, p_r, g1, dp_t, b1, W2)


def _tc_final(q, g2, dp_t, b2):
    grid = (N // BLK,)
    return pl.pallas_call(
        _tc_final_body,
        grid=grid,
        in_specs=[
            pl.BlockSpec((NSC, BLK, N_CLS), lambda i: (0, i, 0)),
            pl.BlockSpec((BLK, 2 * N_CLS), lambda i: (i, 0)),
            pl.BlockSpec((BLK, 2), lambda i: (i, 0)),
            pl.BlockSpec((1, N_CLS), lambda i: (0, 0)),
        ],
        out_specs=pl.BlockSpec((BLK, N_CLS), lambda i: (i, 0)),
        out_shape=jax.ShapeDtypeStruct((N, N_CLS), jnp.float32),
    )(q, g2, dp_t, b2)


def kernel(x, edge_index, edge_weight, W1, b1, W2, b2):
    src_r = edge_index[0].reshape(NW, NCH, C)
    dst_r = edge_index[1].reshape(NW, NCH, C)
    ew_r = edge_weight.reshape(NW, NCH, C)
    ew_flat = edge_weight.reshape(NW, EPT)

    deg_parts = _sc_deg(dst_r, ew_r)
    dp_t = deg_parts[:, :N].T

    g1 = _tc_mm1(x, W1, dp_t)
    p_l = _sc_prop(g1, src_r, dst_r, ew_flat, 0)
    p_r = _sc_prop(g1, src_r, dst_r, ew_flat, 1)
    g2p = _tc_mid(p_l[:, :N, :], p_r[:, :N, :], g1, dp_t,
                  b1.reshape(1, -1), W2)
    q = _sc_prop(g2p, src_r, dst_r, ew_flat, 0)
    out = _tc_final(q[:, :N, :], g2p, dp_t, b2.reshape(1, -1))
    return out


# speedup vs baseline: 1.3800x; 1.3800x over previous
import functools

import jax
import jax.numpy as jnp
from jax import lax
from jax.experimental import pallas as pl
from jax.experimental.pallas import tpu as pltpu
from jax.experimental.pallas import tpu_sc as plsc

N = 10000
E = 320000
D_IN = 128
D_HID = 128
N_CLS = 64

L = 16
NSC = 2
NTILE = 16
NW = NSC * NTILE
EPT = E // NW
C = 80
NCH = EPT // C
NPAD = 10240
RPT = NPAD // NTILE

_mesh = plsc.VectorSubcoreMesh(core_axis_name="c", subcore_axis_name="s")
_sc_params = pltpu.CompilerParams(needs_layout_passes=False,
                                  use_tc_tiling_on_sc=False)


def _sc_deg_body(dst_hbm, ew_hbm, out_hbm, dst_v, ew_v, acc, zbuf):
    c = lax.axis_index("c")
    s = lax.axis_index("s")
    wid = s * NSC + c
    pltpu.sync_copy(dst_hbm.at[wid], dst_v)
    pltpu.sync_copy(ew_hbm.at[wid], ew_v)

    zeros = jnp.zeros((L,), jnp.float32)

    def zb(i, _):
        zbuf[pl.ds(i * L, L)] = zeros
        return 0

    lax.fori_loop(0, RPT // L, zb, 0)
    pltpu.sync_copy(zbuf, acc.at[pl.ds(s * RPT, RPT)])
    plsc.subcore_barrier()

    def chunk(j, _):
        pltpu.sync_copy(ew_v.at[j], acc.at[dst_v.at[j]], add=True)
        return 0

    lax.fori_loop(0, NCH, chunk, 0)
    plsc.subcore_barrier()
    pltpu.sync_copy(acc.at[pl.ds(s * RPT, RPT)],
                    out_hbm.at[c].at[pl.ds(s * RPT, RPT)])


def _sc_deg(dst_r, ew_r):
    return pl.kernel(
        _sc_deg_body,
        out_type=jax.ShapeDtypeStruct((NSC, NPAD), jnp.float32),
        mesh=_mesh,
        scratch_types=[
            pltpu.VMEM((NCH, C), jnp.int32),
            pltpu.VMEM((NCH, C), jnp.float32),
            pltpu.VMEM_SHARED((NPAD,), jnp.float32),
            pltpu.VMEM((RPT,), jnp.float32),
        ],
        compiler_params=_sc_params,
    )(dst_r, ew_r)


def _sc_prop_body(D, h, g_hbm, src_hbm, dst_hbm, ewb_hbm, out_hbm,
                  src_v, db0, db1, eb0, eb1, rows0, rows1, sc0, acc,
                  sem0, sem1, isem0, isem1):
    c = lax.axis_index("c")
    s = lax.axis_index("s")
    wid = s * NSC + c
    pltpu.sync_copy(src_hbm.at[wid], src_v)

    zeros = jnp.zeros((L,), jnp.float32)

    def zrow(r, _):
        for k in range(D // L):
            sc0[r, pl.ds(k * L, L)] = zeros
        return 0

    lax.fori_loop(0, C, zrow, 0)
    for i in range(RPT // C):
        pltpu.sync_copy(sc0, acc.at[pl.ds(s * RPT + i * C, C)])
    plsc.subcore_barrier()

    CB = C // 8
    ebase = wid * (EPT // 8)

    def scale(rows, eb):
        def group(t, _):
            for i in range(L):
                e = t * L + i
                ewb = eb[2 * t + i // 8, pl.ds((i % 8) * L, L)]
                for k in range(D // L):
                    sc0[e, pl.ds(k * L, L)] = \
                        rows[e, pl.ds(h * 64 + k * L, L)] * ewb
            return 0

        lax.fori_loop(0, C // L, group, 0)

    def gather(j, rows, eb, sem):
        pltpu.async_copy(g_hbm.at[src_v.at[j]], rows, sem)
        pltpu.async_copy(ewb_hbm.at[pl.ds(ebase + j * CB, CB)], eb, sem)

    def gwait(j, rows, eb, sem):
        pltpu.make_async_copy(g_hbm.at[src_v.at[j]], rows, sem).wait()
        pltpu.make_async_copy(ewb_hbm.at[pl.ds(ebase + j * CB, CB)], eb,
                              sem).wait()

    def dref(j):
        return dst_hbm.at[wid].at[pl.ds(j * C, C)]

    def istart(j, db, isem):
        pltpu.async_copy(dref(j), db, isem)

    def iwait(j, db, isem):
        pltpu.make_async_copy(dref(j), db, isem).wait()

    gather(0, rows0, eb0, sem0)
    istart(0, db0, isem0)

    def step(j, rows, eb, sem, nrows, neb, nsem, db, isem, ndb, nisem,
             last):
        gwait(j, rows, eb, sem)
        if not last:
            gather(j + 1, nrows, neb, nsem)
        iwait(j, db, isem)
        if not last:
            istart(j + 1, ndb, nisem)
        scale(rows, eb)
        pltpu.sync_copy(sc0, acc.at[db], add=True)

    def chunk2(jj, _):
        for b in range(2):
            j = jj * 2 + b
            rows, eb, sem, db, isem = (
                (rows0, eb0, sem0, db0, isem0) if b == 0
                else (rows1, eb1, sem1, db1, isem1))
            nrows, neb, nsem, ndb, nisem = (
                (rows1, eb1, sem1, db1, isem1) if b == 0
                else (rows0, eb0, sem0, db0, isem0))
            step(j, rows, eb, sem, nrows, neb, nsem, db, isem, ndb, nisem,
                 False)
        return 0

    lax.fori_loop(0, NCH // 2, chunk2, 0)
    jt = NCH - 1
    step(jt, rows0, eb0, sem0, rows1, eb1, sem1, db0, isem0, db1, isem1,
         True)

    plsc.subcore_barrier()
    for i in range(RPT // C):
        sl = pl.ds(s * RPT + i * C, C)
        pltpu.sync_copy(acc.at[sl], out_hbm.at[c].at[sl])


def _sc_prop(g, src_r, dst_flat, ewb, D, h):
    return pl.kernel(
        functools.partial(_sc_prop_body, D, h),
        out_type=jax.ShapeDtypeStruct((NSC, NPAD, D), jnp.float32),
        mesh=_mesh,
        scratch_types=[
            pltpu.VMEM((NCH, C), jnp.int32),
            pltpu.VMEM((C,), jnp.int32),
            pltpu.VMEM((C,), jnp.int32),
            pltpu.VMEM((C // 8, 128), jnp.float32),
            pltpu.VMEM((C // 8, 128), jnp.float32),
            pltpu.VMEM((C, 128), jnp.float32),
            pltpu.VMEM((C, 128), jnp.float32),
            pltpu.VMEM((C, D), jnp.float32),
            pltpu.VMEM_SHARED((NPAD, D), jnp.float32),
            pltpu.SemaphoreType.DMA,
            pltpu.SemaphoreType.DMA,
            pltpu.SemaphoreType.DMA,
            pltpu.SemaphoreType.DMA,
        ],
        compiler_params=_sc_params,
    )(g, src_r, dst_flat, ewb)


BLK = 1000


def _dis_from(dp_ref):
    return lax.rsqrt(1.0 + dp_ref[:, 0:1] + dp_ref[:, 1:2])


def _tc_mm1_body(x_ref, w_ref, dp_ref, g_ref):
    dis = _dis_from(dp_ref)
    h = jnp.dot(x_ref[...], w_ref[...], preferred_element_type=jnp.float32)
    g_ref[...] = h * dis


def _tc_mid_body(p_ref, g1_ref, dp_ref, b1_ref, w2_ref, g2_ref):
    dis = _dis_from(dp_ref)
    m = p_ref[0, :, :] + p_ref[1, :, :]
    z = dis * (m + g1_ref[...]) + b1_ref[...]
    z = jnp.maximum(z, 0.0)
    h2 = jnp.dot(z, w2_ref[...], preferred_element_type=jnp.float32)
    g2_ref[...] = jnp.concatenate(
        [h2 * dis, jnp.zeros_like(h2)], axis=1)


def _tc_final_body(q_ref, g2_ref, dp_ref, b2_ref, o_ref):
    dis = _dis_from(dp_ref)
    o_ref[...] = dis * (q_ref[0, :, :] + q_ref[1, :, :]
                        + g2_ref[:, :N_CLS]) + b2_ref[...]


def _tc_mm1(x, W1, dp_t):
    grid = (N // BLK,)
    return pl.pallas_call(
        _tc_mm1_body,
        grid=grid,
        in_specs=[
            pl.BlockSpec((BLK, D_IN), lambda i: (i, 0)),
            pl.BlockSpec((D_IN, D_HID), lambda i: (0, 0)),
            pl.BlockSpec((BLK, 2), lambda i: (i, 0)),
        ],
        out_specs=pl.BlockSpec((BLK, D_HID), lambda i: (i, 0)),
        out_shape=jax.ShapeDtypeStruct((N, D_HID), jnp.float32),
    )(x, W1, dp_t)


def _tc_mid(p, g1, dp_t, b1, W2):
    grid = (N // BLK,)
    return pl.pallas_call(
        _tc_mid_body,
        grid=grid,
        in_specs=[
            pl.BlockSpec((NSC, BLK, D_HID), lambda i: (0, i, 0)),
            pl.BlockSpec((BLK, D_HID), lambda i: (i, 0)),
            pl.BlockSpec((BLK, 2), lambda i: (i, 0)),
            pl.BlockSpec((1, D_HID), lambda i: (0, 0)),
            pl.BlockSpec((D_HID, N_CLS), lambda i: (0, 0)),
        ],
        out_specs=pl.BlockSpec((BLK, 2 * N_CLS), lambda i: (i, 0)),
        out_shape=jax.ShapeDtypeStruct((N, 2 * N_CLS), jnp.float32),
    )(p, g1, dp_t, b1, W2)


def _tc_final(q, g2, dp_t, b2):
    grid = (N // BLK,)
    return pl.pallas_call(
        _tc_final_body,
        grid=grid,
        in_specs=[
            pl.BlockSpec((NSC, BLK, N_CLS), lambda i: (0, i, 0)),
            pl.BlockSpec((BLK, 2 * N_CLS), lambda i: (i, 0)),
            pl.BlockSpec((BLK, 2), lambda i: (i, 0)),
            pl.BlockSpec((1, N_CLS), lambda i: (0, 0)),
        ],
        out_specs=pl.BlockSpec((BLK, N_CLS), lambda i: (i, 0)),
        out_shape=jax.ShapeDtypeStruct((N, N_CLS), jnp.float32),
    )(q, g2, dp_t, b2)


def kernel(x, edge_index, edge_weight, W1, b1, W2, b2):
    src_r = edge_index[0].reshape(NW, NCH, C)
    dst_r = edge_index[1].reshape(NW, NCH, C)
    ew_r = edge_weight.reshape(NW, NCH, C)
    ewb = jnp.broadcast_to(edge_weight[:, None], (E, L)).reshape(E // 8, 8 * L)

    deg_parts = _sc_deg(dst_r, ew_r)
    dp_t = deg_parts[:, :N].T

    g1 = _tc_mm1(x, W1, dp_t)
    dst_flat = edge_index[1].reshape(NW, EPT)
    p = _sc_prop(g1, src_r, dst_flat, ewb, D_HID, 0)
    g2p = _tc_mid(p[:, :N, :], g1, dp_t,
                  b1.reshape(1, -1), W2)
    q = _sc_prop(g2p, src_r, dst_flat, ewb, N_CLS, 0)
    out = _tc_final(q[:, :N, :], g2p, dp_t, b2.reshape(1, -1))
    return out


# speedup vs baseline: 1.6090x; 1.1659x over previous
import functools

import jax
import jax.numpy as jnp
from jax import lax
from jax.experimental import pallas as pl
from jax.experimental.pallas import tpu as pltpu
from jax.experimental.pallas import tpu_sc as plsc

N = 10000
E = 320000
D_IN = 128
D_HID = 128
N_CLS = 64

L = 16
NSC = 2
NTILE = 16
NW = NSC * NTILE
EPT = E // NW
C = 80
NCH = EPT // C
NPAD = 10240
RPT = NPAD // NTILE

_mesh = plsc.VectorSubcoreMesh(core_axis_name="c", subcore_axis_name="s")
_sc_params = pltpu.CompilerParams(needs_layout_passes=False,
                                  use_tc_tiling_on_sc=False)


def _sc_deg_body(dst_hbm, ew_hbm, out_hbm, dst_v, ew_v, acc, zbuf):
    c = lax.axis_index("c")
    s = lax.axis_index("s")
    wid = s * NSC + c
    pltpu.sync_copy(dst_hbm.at[wid], dst_v)
    pltpu.sync_copy(ew_hbm.at[wid], ew_v)

    zeros = jnp.zeros((L,), jnp.float32)

    def zb(i, _):
        zbuf[pl.ds(i * L, L)] = zeros
        return 0

    lax.fori_loop(0, RPT // L, zb, 0)
    pltpu.sync_copy(zbuf, acc.at[pl.ds(s * RPT, RPT)])
    plsc.subcore_barrier()

    def chunk(j, _):
        pltpu.sync_copy(ew_v.at[j], acc.at[dst_v.at[j]], add=True)
        return 0

    lax.fori_loop(0, NCH, chunk, 0)
    plsc.subcore_barrier()
    pltpu.sync_copy(acc.at[pl.ds(s * RPT, RPT)],
                    out_hbm.at[c].at[pl.ds(s * RPT, RPT)])


def _sc_deg(dst_r, ew_r):
    return pl.kernel(
        _sc_deg_body,
        out_type=jax.ShapeDtypeStruct((NSC, NPAD), jnp.float32),
        mesh=_mesh,
        scratch_types=[
            pltpu.VMEM((NCH, C), jnp.int32),
            pltpu.VMEM((NCH, C), jnp.float32),
            pltpu.VMEM_SHARED((NPAD,), jnp.float32),
            pltpu.VMEM((RPT,), jnp.float32),
        ],
        compiler_params=_sc_params,
    )(dst_r, ew_r)


def _sc_prop_body(D, h, g_hbm, src_hbm, dst_hbm, ewb_hbm, out_hbm,
                  src_v, db0, db1, eb0, eb1, rows0, rows1, sc0, acc,
                  sem0, sem1, isem0, isem1):
    c = lax.axis_index("c")
    s = lax.axis_index("s")
    wid = s * NSC + c
    pltpu.sync_copy(src_hbm.at[wid], src_v)

    zeros = jnp.zeros((L,), jnp.float32)

    def zrow(r, _):
        for k in range(D // L):
            sc0[r, pl.ds(k * L, L)] = zeros
        return 0

    lax.fori_loop(0, C, zrow, 0)
    for i in range(RPT // C):
        pltpu.sync_copy(sc0, acc.at[pl.ds(s * RPT + i * C, C)])
    plsc.subcore_barrier()

    CB = C // 8
    ebase = wid * (EPT // 8)

    def scale(rows, eb):
        def group(t, _):
            for i in range(L):
                e = t * L + i
                ewb = eb[2 * t + i // 8, pl.ds((i % 8) * L, L)]
                for k in range(D // L):
                    sc0[e, pl.ds(k * L, L)] = \
                        rows[e, pl.ds(h * 64 + k * L, L)] * ewb
            return 0

        lax.fori_loop(0, C // L, group, 0)

    def gather(j, rows, eb, sem):
        pltpu.async_copy(g_hbm.at[src_v.at[j]], rows, sem)
        pltpu.async_copy(ewb_hbm.at[pl.ds(ebase + j * CB, CB)], eb, sem)

    def gwait(j, rows, eb, sem):
        pltpu.make_async_copy(g_hbm.at[src_v.at[j]], rows, sem).wait()
        pltpu.make_async_copy(ewb_hbm.at[pl.ds(ebase + j * CB, CB)], eb,
                              sem).wait()

    def dref(j):
        return dst_hbm.at[wid].at[pl.ds(j * C, C)]

    def istart(j, db, isem):
        pltpu.async_copy(dref(j), db, isem)

    def iwait(j, db, isem):
        pltpu.make_async_copy(dref(j), db, isem).wait()

    gather(0, rows0, eb0, sem0)
    istart(0, db0, isem0)

    def step(j, rows, eb, sem, nrows, neb, nsem, db, isem, ndb, nisem,
             last):
        gwait(j, rows, eb, sem)
        if not last:
            gather(j + 1, nrows, neb, nsem)
        iwait(j, db, isem)
        if not last:
            istart(j + 1, ndb, nisem)
        scale(rows, eb)
        pltpu.sync_copy(sc0, acc.at[db], add=True)

    def chunk2(jj, _):
        for b in range(2):
            j = jj * 2 + b
            rows, eb, sem, db, isem = (
                (rows0, eb0, sem0, db0, isem0) if b == 0
                else (rows1, eb1, sem1, db1, isem1))
            nrows, neb, nsem, ndb, nisem = (
                (rows1, eb1, sem1, db1, isem1) if b == 0
                else (rows0, eb0, sem0, db0, isem0))
            step(j, rows, eb, sem, nrows, neb, nsem, db, isem, ndb, nisem,
                 False)
        return 0

    lax.fori_loop(0, NCH // 2, chunk2, 0)
    jt = NCH - 1
    step(jt, rows0, eb0, sem0, rows1, eb1, sem1, db0, isem0, db1, isem1,
         True)

    plsc.subcore_barrier()
    for i in range(RPT // C):
        sl = pl.ds(s * RPT + i * C, C)
        pltpu.sync_copy(acc.at[sl], out_hbm.at[c].at[sl])


def _sc_prop(g, src_r, dst_flat, ewb, D, h):
    return pl.kernel(
        functools.partial(_sc_prop_body, D, h),
        out_type=jax.ShapeDtypeStruct((NSC, NPAD, D), jnp.float32),
        mesh=_mesh,
        scratch_types=[
            pltpu.VMEM((NCH, C), jnp.int32),
            pltpu.VMEM((C,), jnp.int32),
            pltpu.VMEM((C,), jnp.int32),
            pltpu.VMEM((C // 8, 128), jnp.float32),
            pltpu.VMEM((C // 8, 128), jnp.float32),
            pltpu.VMEM((C, 128), jnp.float32),
            pltpu.VMEM((C, 128), jnp.float32),
            pltpu.VMEM((C, D), jnp.float32),
            pltpu.VMEM_SHARED((NPAD, D), jnp.float32),
            pltpu.SemaphoreType.DMA,
            pltpu.SemaphoreType.DMA,
            pltpu.SemaphoreType.DMA,
            pltpu.SemaphoreType.DMA,
        ],
        compiler_params=_sc_params,
    )(g, src_r, dst_flat, ewb)


BLK = 1000


def _dis_from(dp_ref):
    return lax.rsqrt(1.0 + dp_ref[:, 0:1] + dp_ref[:, 1:2])


def _tc_mm1_body(x_ref, w_ref, dp_ref, g_ref):
    dis = _dis_from(dp_ref)
    h = jnp.dot(x_ref[...], w_ref[...], preferred_element_type=jnp.float32)
    g_ref[...] = h * dis


def _tc_mid_body(p_ref, g1_ref, dp_ref, b1_ref, w2_ref, g2_ref):
    dis = _dis_from(dp_ref)
    m = p_ref[0, :, :] + p_ref[1, :, :]
    z = dis * (m + g1_ref[...]) + b1_ref[...]
    z = jnp.maximum(z, 0.0)
    h2 = jnp.dot(z, w2_ref[...], preferred_element_type=jnp.float32)
    g2_ref[...] = jnp.concatenate(
        [h2 * dis, jnp.zeros_like(h2)], axis=1)


def _tc_final_body(q_ref, g2_ref, dp_ref, b2_ref, o_ref):
    dis = _dis_from(dp_ref)
    o_ref[...] = dis * (q_ref[0, :, :N_CLS] + q_ref[1, :, :N_CLS]
                        + g2_ref[:, :N_CLS]) + b2_ref[...]


def _tc_mm1(x, W1, dp_t):
    grid = (N // BLK,)
    return pl.pallas_call(
        _tc_mm1_body,
        grid=grid,
        in_specs=[
            pl.BlockSpec((BLK, D_IN), lambda i: (i, 0)),
            pl.BlockSpec((D_IN, D_HID), lambda i: (0, 0)),
            pl.BlockSpec((BLK, 2), lambda i: (i, 0)),
        ],
        out_specs=pl.BlockSpec((BLK, D_HID), lambda i: (i, 0)),
        out_shape=jax.ShapeDtypeStruct((N, D_HID), jnp.float32),
    )(x, W1, dp_t)


def _tc_mid(p, g1, dp_t, b1, W2):
    grid = (N // BLK,)
    return pl.pallas_call(
        _tc_mid_body,
        grid=grid,
        in_specs=[
            pl.BlockSpec((NSC, BLK, D_HID), lambda i: (0, i, 0)),
            pl.BlockSpec((BLK, D_HID), lambda i: (i, 0)),
            pl.BlockSpec((BLK, 2), lambda i: (i, 0)),
            pl.BlockSpec((1, D_HID), lambda i: (0, 0)),
            pl.BlockSpec((D_HID, N_CLS), lambda i: (0, 0)),
        ],
        out_specs=pl.BlockSpec((BLK, 2 * N_CLS), lambda i: (i, 0)),
        out_shape=jax.ShapeDtypeStruct((N, 2 * N_CLS), jnp.float32),
    )(p, g1, dp_t, b1, W2)


def _tc_final(q, g2, dp_t, b2):
    grid = (N // BLK,)
    return pl.pallas_call(
        _tc_final_body,
        grid=grid,
        in_specs=[
            pl.BlockSpec((NSC, BLK, 2 * N_CLS), lambda i: (0, i, 0)),
            pl.BlockSpec((BLK, 2 * N_CLS), lambda i: (i, 0)),
            pl.BlockSpec((BLK, 2), lambda i: (i, 0)),
            pl.BlockSpec((1, N_CLS), lambda i: (0, 0)),
        ],
        out_specs=pl.BlockSpec((BLK, N_CLS), lambda i: (i, 0)),
        out_shape=jax.ShapeDtypeStruct((N, N_CLS), jnp.float32),
    )(q, g2, dp_t, b2)


def kernel(x, edge_index, edge_weight, W1, b1, W2, b2):
    src_r = edge_index[0].reshape(NW, NCH, C)
    dst_r = edge_index[1].reshape(NW, NCH, C)
    ew_r = edge_weight.reshape(NW, NCH, C)
    ewb = jnp.broadcast_to(edge_weight[:, None], (E, L)).reshape(E // 8, 8 * L)

    deg_parts = _sc_deg(dst_r, ew_r)
    dp_t = deg_parts[:, :N].T

    g1 = _tc_mm1(x, W1, dp_t)
    dst_flat = edge_index[1].reshape(NW, EPT)
    p = _sc_prop(g1, src_r, dst_flat, ewb, D_HID, 0)
    g2p = _tc_mid(p[:, :N, :], g1, dp_t,
                  b1.reshape(1, -1), W2)
    q = _sc_prop(g2p, src_r, dst_flat, ewb, D_HID, 0)
    out = _tc_final(q[:, :N, :], g2p, dp_t, b2.reshape(1, -1))
    return out
